# Initial kernel scaffold; baseline (speedup 1.0000x reference)
import jax
import jax.numpy as jnp
from jax.experimental import pallas as pl


def kernel(x, edge_index, W, b):
    raise NotImplementedError("write your pallas kernel here")


# speedup vs baseline: 21.9958x; 21.9958x over previous
import functools

import jax
import jax.numpy as jnp
from jax import lax
from jax.experimental import pallas as pl
from jax.experimental.pallas import tpu as pltpu
from jax.experimental.pallas import tpu_sc as plsc

NC = 2
NS = 16
NW = NC * NS
B = 128
C = 1024


def _pick_k(total):
    for k in range(56, 0, -1):
        if total % k == 0:
            return k
    return 1


def _sc_histogram(dstp, zeros1, n_pad, G, K):
    rows_per_tile = n_pad // NS
    mesh = plsc.VectorSubcoreMesh(core_axis_name="c", subcore_axis_name="s")

    @functools.partial(
        pl.kernel,
        out_type=jax.ShapeDtypeStruct((NC, n_pad), jnp.float32),
        mesh=mesh,
        scratch_types=[
            pltpu.VMEM_SHARED((n_pad,), jnp.float32),
            pltpu.VMEM((K, B), jnp.int32),
            pltpu.VMEM((B,), jnp.float32),
        ],
    )
    def hist_kernel(dst_hbm, zeros_hbm, deg_out, deg_sp, dst_buf, ones_v):
        c = lax.axis_index("c")
        s = lax.axis_index("s")
        wid = c * NS + s
        sl = pl.ds(s * rows_per_tile, rows_per_tile)
        pltpu.sync_copy(zeros_hbm.at[sl], deg_sp.at[sl])
        for i in range(B // 16):
            ones_v[pl.ds(i * 16, 16)] = jnp.full((16,), 1.0, jnp.float32)
        plsc.subcore_barrier()

        def outer(g, carry):
            pltpu.sync_copy(dst_hbm.at[wid, g], dst_buf)

            def inner(j, carry2):
                pltpu.sync_copy(ones_v, deg_sp.at[dst_buf.at[j]], add=True)
                return carry2

            return lax.fori_loop(0, K, inner, carry)

        lax.fori_loop(0, G, outer, 0)
        plsc.subcore_barrier()
        pltpu.sync_copy(deg_sp.at[sl], deg_out.at[c, sl])

    return hist_kernel(dstp, zeros1)


def _sc_edges(srcp, dstp, wcols, zeros1, n_pad, G, K):
    rows_per_tile = n_pad // NS
    mesh = plsc.VectorSubcoreMesh(core_axis_name="c", subcore_axis_name="s")

    @functools.partial(
        pl.kernel,
        out_type=jax.ShapeDtypeStruct((NC, 4, n_pad), jnp.float32),
        mesh=mesh,
        scratch_types=[
            pltpu.VMEM_SHARED((n_pad,), jnp.float32),
            pltpu.VMEM_SHARED((n_pad,), jnp.float32),
            pltpu.VMEM_SHARED((n_pad,), jnp.float32),
            pltpu.VMEM_SHARED((n_pad,), jnp.float32),
            pltpu.VMEM((K, B), jnp.int32),
            pltpu.VMEM((K, B), jnp.int32),
            pltpu.VMEM((B,), jnp.float32),
            pltpu.VMEM((B,), jnp.float32),
            pltpu.VMEM((B,), jnp.float32),
            pltpu.VMEM((B,), jnp.float32),
        ],
    )
    def edge_kernel(src_hbm, dst_hbm, w0, w1, w2, w3, z_hbm, acc_out,
                    a0, a1, a2, a3, src_buf, dst_buf, c0, c1, c2, c3):
        c = lax.axis_index("c")
        s = lax.axis_index("s")
        wid = c * NS + s
        sl = pl.ds(s * rows_per_tile, rows_per_tile)
        for a in (a0, a1, a2, a3):
            pltpu.sync_copy(z_hbm.at[sl], a.at[sl])
        plsc.subcore_barrier()

        def outer(g, carry):
            pltpu.sync_copy(src_hbm.at[wid, g], src_buf)
            pltpu.sync_copy(dst_hbm.at[wid, g], dst_buf)

            def inner(j, carry2):
                si = src_buf.at[j]
                di = dst_buf.at[j]
                for w, a, cv in ((w0, a0, c0), (w1, a1, c1),
                                 (w2, a2, c2), (w3, a3, c3)):
                    pltpu.sync_copy(w.at[si], cv)
                    pltpu.sync_copy(cv, a.at[di], add=True)
                return carry2

            return lax.fori_loop(0, K, inner, carry)

        lax.fori_loop(0, G, outer, 0)
        plsc.subcore_barrier()
        for k, a in enumerate((a0, a1, a2, a3)):
            pltpu.sync_copy(a.at[sl], acc_out.at[c, k, sl])

    return edge_kernel(srcp, dstp, *wcols, zeros1)


def _tc_norm(degp, xt, n_pad):
    def body(degp_ref, xt_ref, wst_ref, dis_ref):
        deg = degp_ref[0, :] + degp_ref[1, :] + 1.0
        dis = lax.rsqrt(deg)
        dis_ref[...] = dis
        wst_ref[...] = dis[None, :] * xt_ref[...]

    return pl.pallas_call(
        body,
        grid=(n_pad // C,),
        in_specs=[
            pl.BlockSpec((NC, C), lambda i: (0, i)),
            pl.BlockSpec((4, C), lambda i: (0, i)),
        ],
        out_specs=[
            pl.BlockSpec((4, C), lambda i: (0, i)),
            pl.BlockSpec((C,), lambda i: (i,)),
        ],
        out_shape=[
            jax.ShapeDtypeStruct((4, n_pad), jnp.float32),
            jax.ShapeDtypeStruct((n_pad,), jnp.float32),
        ],
    )(degp, xt)


def _tc_final(accp, wst, dis, W, b, n, n_pad):
    def body(accp_ref, wst_ref, dis_ref, w_ref, b_ref, out_ref):
        st = accp_ref[0] + accp_ref[1] + wst_ref[...]
        pret = dis_ref[...][None, :] * st
        out_ref[...] = (
            lax.dot_general(pret, w_ref[...], (((0,), (0,)), ((), ())),
                            preferred_element_type=jnp.float32)
            + b_ref[...][None, :]
        )

    return pl.pallas_call(
        body,
        grid=(pl.cdiv(n, C),),
        in_specs=[
            pl.BlockSpec((NC, 4, C), lambda i: (0, 0, i)),
            pl.BlockSpec((4, C), lambda i: (0, i)),
            pl.BlockSpec((C,), lambda i: (i,)),
            pl.BlockSpec((4, 128), lambda i: (0, 0)),
            pl.BlockSpec((128,), lambda i: (0,)),
        ],
        out_specs=pl.BlockSpec((C, 128), lambda i: (i, 0)),
        out_shape=jax.ShapeDtypeStruct((n, W.shape[1]), jnp.float32),
    )(accp, wst, dis, W, b)


def kernel(x, edge_index, W, b):
    n = x.shape[0]
    e = edge_index.shape[1]
    n_pad = pl.cdiv(n + 1, C) * C
    per_w = pl.cdiv(e, NW * B) * B
    total = per_w // B
    K = _pick_k(total)
    G = total // K
    pad_e = NW * per_w - e

    fill = jnp.full((pad_e,), n, dtype=jnp.int32)
    srcp = jnp.concatenate([edge_index[0], fill]).reshape(NW, G, K, B)
    dstp = jnp.concatenate([edge_index[1], fill]).reshape(NW, G, K, B)
    xt = jnp.zeros((4, n_pad), jnp.float32).at[:, :n].set(x.T)
    zeros1 = jnp.zeros((n_pad,), jnp.float32)

    degp = _sc_histogram(dstp, zeros1, n_pad, G, K)
    wst, dis = _tc_norm(degp, xt, n_pad)
    wcols = [wst[k] for k in range(4)]
    accp = _sc_edges(srcp, dstp, wcols, zeros1, n_pad, G, K)
    return _tc_final(accp, wst, dis, W, b, n, n_pad)


# speedup vs baseline: 73.3268x; 3.3337x over previous
import functools

import jax
import jax.numpy as jnp
from jax import lax
from jax.experimental import pallas as pl
from jax.experimental.pallas import tpu as pltpu
from jax.experimental.pallas import tpu_sc as plsc

NC = 2
NS = 16
NW = NC * NS
B = 128
C = 1024


def _pick_k(total):
    for k in range(56, 0, -1):
        if total % k == 0:
            return k
    return 1


def _sc_histogram(dstp, zeros1, ones_kb, n_pad, G, KB):
    rows_per_tile = n_pad // NS
    mesh = plsc.VectorSubcoreMesh(core_axis_name="c", subcore_axis_name="s")

    @functools.partial(
        pl.kernel,
        out_type=jax.ShapeDtypeStruct((NC, n_pad), jnp.float32),
        mesh=mesh,
        scratch_types=[
            pltpu.VMEM_SHARED((n_pad,), jnp.float32),
            pltpu.VMEM((KB,), jnp.int32),
            pltpu.VMEM((KB,), jnp.float32),
        ],
    )
    def hist_kernel(dst_hbm, zeros_hbm, ones_hbm, deg_out,
                    deg_sp, dst_buf, ones_v):
        c = lax.axis_index("c")
        s = lax.axis_index("s")
        wid = c * NS + s
        sl = pl.ds(s * rows_per_tile, rows_per_tile)
        pltpu.sync_copy(zeros_hbm.at[sl], deg_sp.at[sl])
        pltpu.sync_copy(ones_hbm, ones_v)
        plsc.subcore_barrier()

        def outer(g, carry):
            pltpu.sync_copy(dst_hbm.at[wid, g], dst_buf)
            pltpu.sync_copy(ones_v, deg_sp.at[dst_buf], add=True)
            return carry

        lax.fori_loop(0, G, outer, 0)
        plsc.subcore_barrier()
        pltpu.sync_copy(deg_sp.at[sl], deg_out.at[c, sl])

    return hist_kernel(dstp, zeros1, ones_kb)


def _sc_edges(srcp, dstp, wcols, zeros1, n_pad, G, KB):
    rows_per_tile = n_pad // NS
    mesh = plsc.VectorSubcoreMesh(core_axis_name="c", subcore_axis_name="s")

    @functools.partial(
        pl.kernel,
        out_type=jax.ShapeDtypeStruct((NC, 4, n_pad), jnp.float32),
        mesh=mesh,
        scratch_types=[
            pltpu.VMEM_SHARED((n_pad,), jnp.float32),
            pltpu.VMEM_SHARED((n_pad,), jnp.float32),
            pltpu.VMEM_SHARED((n_pad,), jnp.float32),
            pltpu.VMEM_SHARED((n_pad,), jnp.float32),
            pltpu.VMEM_SHARED((n_pad,), jnp.float32),
            pltpu.VMEM_SHARED((n_pad,), jnp.float32),
            pltpu.VMEM_SHARED((n_pad,), jnp.float32),
            pltpu.VMEM_SHARED((n_pad,), jnp.float32),
            pltpu.VMEM((KB,), jnp.int32),
            pltpu.VMEM((KB,), jnp.int32),
            pltpu.VMEM((KB,), jnp.float32),
        ],
    )
    def edge_kernel(src_hbm, dst_hbm, w0, w1, w2, w3, z_hbm, acc_out,
                    a0, a1, a2, a3, w0_sp, w1_sp, w2_sp, w3_sp,
                    src_buf, dst_buf, cv):
        c = lax.axis_index("c")
        s = lax.axis_index("s")
        wid = c * NS + s
        sl = pl.ds(s * rows_per_tile, rows_per_tile)
        for a in (a0, a1, a2, a3):
            pltpu.sync_copy(z_hbm.at[sl], a.at[sl])
        for w, w_sp in ((w0, w0_sp), (w1, w1_sp), (w2, w2_sp), (w3, w3_sp)):
            pltpu.sync_copy(w.at[sl], w_sp.at[sl])
        plsc.subcore_barrier()

        def outer(g, carry):
            pltpu.sync_copy(src_hbm.at[wid, g], src_buf)
            pltpu.sync_copy(dst_hbm.at[wid, g], dst_buf)
            for w_sp, a in ((w0_sp, a0), (w1_sp, a1), (w2_sp, a2), (w3_sp, a3)):
                pltpu.sync_copy(w_sp.at[src_buf], cv)
                pltpu.sync_copy(cv, a.at[dst_buf], add=True)
            return carry

        lax.fori_loop(0, G, outer, 0)
        plsc.subcore_barrier()
        for k, a in enumerate((a0, a1, a2, a3)):
            pltpu.sync_copy(a.at[sl], acc_out.at[c, k, sl])

    return edge_kernel(srcp, dstp, *wcols, zeros1)


def _tc_norm(degp, xt, n_pad):
    def body(degp_ref, xt_ref, wst_ref, dis_ref):
        deg = degp_ref[0, :] + degp_ref[1, :] + 1.0
        dis = lax.rsqrt(deg)
        dis_ref[...] = dis
        wst_ref[...] = dis[None, :] * xt_ref[...]

    return pl.pallas_call(
        body,
        grid=(n_pad // C,),
        in_specs=[
            pl.BlockSpec((NC, C), lambda i: (0, i)),
            pl.BlockSpec((4, C), lambda i: (0, i)),
        ],
        out_specs=[
            pl.BlockSpec((4, C), lambda i: (0, i)),
            pl.BlockSpec((C,), lambda i: (i,)),
        ],
        out_shape=[
            jax.ShapeDtypeStruct((4, n_pad), jnp.float32),
            jax.ShapeDtypeStruct((n_pad,), jnp.float32),
        ],
    )(degp, xt)


def _tc_final(accp, wst, dis, W, b, n, n_pad):
    def body(accp_ref, wst_ref, dis_ref, w_ref, b_ref, out_ref):
        st = accp_ref[0] + accp_ref[1] + wst_ref[...]
        pret = dis_ref[...][None, :] * st
        out_ref[...] = (
            lax.dot_general(pret, w_ref[...], (((0,), (0,)), ((), ())),
                            preferred_element_type=jnp.float32)
            + b_ref[...][None, :]
        )

    return pl.pallas_call(
        body,
        grid=(pl.cdiv(n, C),),
        in_specs=[
            pl.BlockSpec((NC, 4, C), lambda i: (0, 0, i)),
            pl.BlockSpec((4, C), lambda i: (0, i)),
            pl.BlockSpec((C,), lambda i: (i,)),
            pl.BlockSpec((4, 128), lambda i: (0, 0)),
            pl.BlockSpec((128,), lambda i: (0,)),
        ],
        out_specs=pl.BlockSpec((C, 128), lambda i: (i, 0)),
        out_shape=jax.ShapeDtypeStruct((n, W.shape[1]), jnp.float32),
    )(accp, wst, dis, W, b)


def kernel(x, edge_index, W, b):
    n = x.shape[0]
    e = edge_index.shape[1]
    n_pad = pl.cdiv(n + 1, C) * C
    per_w = pl.cdiv(e, NW * B) * B
    total = per_w // B
    K = _pick_k(total)
    G = total // K
    KB = K * B
    pad_e = NW * per_w - e

    fill = jnp.full((pad_e,), n, dtype=jnp.int32)
    srcp = jnp.concatenate([edge_index[0], fill]).reshape(NW, G, KB)
    dstp = jnp.concatenate([edge_index[1], fill]).reshape(NW, G, KB)
    xt = jnp.zeros((4, n_pad), jnp.float32).at[:, :n].set(x.T)
    zeros1 = jnp.zeros((n_pad,), jnp.float32)
    ones_kb = jnp.ones((KB,), jnp.float32)

    degp = _sc_histogram(dstp, zeros1, ones_kb, n_pad, G, KB)
    wst, dis = _tc_norm(degp, xt, n_pad)
    wcols = [wst[k] for k in range(4)]
    accp = _sc_edges(srcp, dstp, wcols, zeros1, n_pad, G, KB)
    return _tc_final(accp, wst, dis, W, b, n, n_pad)


# speedup vs baseline: 79.9585x; 1.0904x over previous
import functools

import jax
import jax.numpy as jnp
from jax import lax
from jax.experimental import pallas as pl
from jax.experimental.pallas import tpu as pltpu
from jax.experimental.pallas import tpu_sc as plsc

NC = 2
NS = 16
NW = NC * NS
B = 128
C = 1024
L = 16


def _pick_k(total):
    for k in range(56, 0, -1):
        if total % k == 0:
            return k
    return 1


def _sc_histogram(dstp, zeros1, ones_kb, n_pad, G, KB):
    rows_per_tile = n_pad // NS
    mesh = plsc.VectorSubcoreMesh(core_axis_name="c", subcore_axis_name="s")

    @functools.partial(
        pl.kernel,
        out_type=jax.ShapeDtypeStruct((NC, n_pad), jnp.float32),
        mesh=mesh,
        scratch_types=[
            pltpu.VMEM_SHARED((n_pad,), jnp.float32),
            pltpu.VMEM((KB,), jnp.int32),
            pltpu.VMEM((KB,), jnp.float32),
        ],
    )
    def hist_kernel(dst_hbm, zeros_hbm, ones_hbm, deg_out,
                    deg_sp, dst_buf, ones_v):
        c = lax.axis_index("c")
        s = lax.axis_index("s")
        wid = c * NS + s
        sl = pl.ds(s * rows_per_tile, rows_per_tile)
        pltpu.sync_copy(zeros_hbm.at[sl], deg_sp.at[sl])
        pltpu.sync_copy(ones_hbm, ones_v)
        plsc.subcore_barrier()

        def outer(g, carry):
            pltpu.sync_copy(dst_hbm.at[wid, g], dst_buf)
            pltpu.sync_copy(ones_v, deg_sp.at[dst_buf], add=True)
            return carry

        lax.fori_loop(0, G, outer, 0)
        plsc.subcore_barrier()
        pltpu.sync_copy(deg_sp.at[sl], deg_out.at[c, sl])

    return hist_kernel(dstp, zeros1, ones_kb)


def _sc_edges(srcp, dstp, degp, xt, zeros1, n_pad, G, KB):
    rows_per_tile = n_pad // NS
    mesh = plsc.VectorSubcoreMesh(core_axis_name="c", subcore_axis_name="s")

    @functools.partial(
        pl.kernel,
        out_type=jax.ShapeDtypeStruct((NC, 4, n_pad), jnp.float32),
        mesh=mesh,
        compiler_params=pltpu.CompilerParams(needs_layout_passes=False),
        scratch_types=[
            pltpu.VMEM_SHARED((n_pad,), jnp.float32),
            pltpu.VMEM_SHARED((n_pad,), jnp.float32),
            pltpu.VMEM_SHARED((n_pad,), jnp.float32),
            pltpu.VMEM_SHARED((n_pad,), jnp.float32),
            pltpu.VMEM_SHARED((n_pad,), jnp.float32),
            pltpu.VMEM_SHARED((n_pad,), jnp.float32),
            pltpu.VMEM_SHARED((n_pad,), jnp.float32),
            pltpu.VMEM_SHARED((n_pad,), jnp.float32),
            pltpu.VMEM((KB,), jnp.int32),
            pltpu.VMEM((KB,), jnp.int32),
            pltpu.VMEM((KB,), jnp.float32),
            pltpu.VMEM((rows_per_tile,), jnp.float32),
            pltpu.VMEM((rows_per_tile,), jnp.float32),
        ],
    )
    def edge_kernel(src_hbm, dst_hbm, degp_hbm, xt_hbm, z_hbm, acc_out,
                    a0, a1, a2, a3, w0_sp, w1_sp, w2_sp, w3_sp,
                    src_buf, dst_buf, cv, disv, xv):
        c = lax.axis_index("c")
        s = lax.axis_index("s")
        wid = c * NS + s
        row0 = s * rows_per_tile
        sl = pl.ds(row0, rows_per_tile)
        for a in (a0, a1, a2, a3):
            pltpu.sync_copy(z_hbm.at[sl], a.at[sl])
        pltpu.sync_copy(degp_hbm.at[0, sl], disv)
        pltpu.sync_copy(degp_hbm.at[1, sl], xv)

        def rsqrt_step(i, carry):
            ds16 = pl.ds(i * L, L)
            deg = disv[ds16] + xv[ds16] + 1.0
            iy = jnp.int32(0x5F3759DF) - lax.shift_right_logical(
                plsc.bitcast(deg, jnp.int32), 1)
            y = plsc.bitcast(iy, jnp.float32)
            y = y * (1.5 - 0.5 * deg * y * y)
            y = y * (1.5 - 0.5 * deg * y * y)
            y = y * (1.5 - 0.5 * deg * y * y)
            disv[ds16] = y
            return carry

        lax.fori_loop(0, rows_per_tile // L, rsqrt_step, 0)
        for k, w_sp in enumerate((w0_sp, w1_sp, w2_sp, w3_sp)):
            pltpu.sync_copy(xt_hbm.at[k, sl], xv)

            def scale_step(i, carry):
                ds16 = pl.ds(i * L, L)
                xv[ds16] = xv[ds16] * disv[ds16]
                return carry

            lax.fori_loop(0, rows_per_tile // L, scale_step, 0)
            pltpu.sync_copy(xv, w_sp.at[sl])
        plsc.subcore_barrier()

        def outer(g, carry):
            pltpu.sync_copy(src_hbm.at[wid, g], src_buf)
            pltpu.sync_copy(dst_hbm.at[wid, g], dst_buf)
            for w_sp, a in ((w0_sp, a0), (w1_sp, a1), (w2_sp, a2), (w3_sp, a3)):
                pltpu.sync_copy(w_sp.at[src_buf], cv)
                pltpu.sync_copy(cv, a.at[dst_buf], add=True)
            return carry

        lax.fori_loop(0, G, outer, 0)
        plsc.subcore_barrier()
        for k, a in enumerate((a0, a1, a2, a3)):
            pltpu.sync_copy(a.at[sl], acc_out.at[c, k, sl])

    return edge_kernel(srcp, dstp, degp, xt, zeros1)


def _tc_final(accp, degp, xt, W, b, n, n_pad):
    def body(accp_ref, degp_ref, xt_ref, w_ref, b_ref, out_ref):
        deg = degp_ref[0, :] + degp_ref[1, :] + 1.0
        dis = lax.rsqrt(deg)
        st = dis[None, :] * (accp_ref[0] + accp_ref[1]) \
            + (dis * dis)[None, :] * xt_ref[...]
        out_ref[...] = (
            lax.dot_general(st, w_ref[...], (((0,), (0,)), ((), ())),
                            preferred_element_type=jnp.float32)
            + b_ref[...][None, :]
        )

    return pl.pallas_call(
        body,
        grid=(pl.cdiv(n, C),),
        in_specs=[
            pl.BlockSpec((NC, 4, C), lambda i: (0, 0, i)),
            pl.BlockSpec((NC, C), lambda i: (0, i)),
            pl.BlockSpec((4, C), lambda i: (0, i)),
            pl.BlockSpec((4, 128), lambda i: (0, 0)),
            pl.BlockSpec((128,), lambda i: (0,)),
        ],
        out_specs=pl.BlockSpec((C, 128), lambda i: (i, 0)),
        out_shape=jax.ShapeDtypeStruct((n, W.shape[1]), jnp.float32),
    )(accp, degp, xt, W, b)


def kernel(x, edge_index, W, b):
    n = x.shape[0]
    e = edge_index.shape[1]
    n_pad = pl.cdiv(n + 1, C) * C
    per_w = pl.cdiv(e, NW * B) * B
    total = per_w // B
    K = _pick_k(total)
    G = total // K
    KB = K * B
    pad_e = NW * per_w - e

    fill = jnp.full((pad_e,), n, dtype=jnp.int32)
    srcp = jnp.concatenate([edge_index[0], fill]).reshape(NW, G, KB)
    dstp = jnp.concatenate([edge_index[1], fill]).reshape(NW, G, KB)
    xt = jnp.zeros((4, n_pad), jnp.float32).at[:, :n].set(x.T)
    zeros1 = jnp.zeros((n_pad,), jnp.float32)
    ones_kb = jnp.ones((KB,), jnp.float32)

    degp = _sc_histogram(dstp, zeros1, ones_kb, n_pad, G, KB)
    accp = _sc_edges(srcp, dstp, degp, xt, zeros1, n_pad, G, KB)
    return _tc_final(accp, degp, xt, W, b, n, n_pad)


# speedup vs baseline: 92.9306x; 1.1622x over previous
import functools

import jax
import jax.numpy as jnp
from jax import lax
from jax.experimental import pallas as pl
from jax.experimental.pallas import tpu as pltpu
from jax.experimental.pallas import tpu_sc as plsc

NC = 2
NS = 16
NW = NC * NS
B = 128
C = 1024
L = 16


def _pick_k(total):
    for k in range(56, 0, -1):
        if total % k == 0:
            return k
    return 1


def _sc_histogram(dstp, zeros1, ones_kb, n_pad, G, KB):
    rows_per_tile = n_pad // NS
    mesh = plsc.VectorSubcoreMesh(core_axis_name="c", subcore_axis_name="s")

    @functools.partial(
        pl.kernel,
        out_type=jax.ShapeDtypeStruct((NC, n_pad), jnp.float32),
        mesh=mesh,
        scratch_types=[
            pltpu.VMEM_SHARED((n_pad,), jnp.float32),
            pltpu.VMEM((KB,), jnp.int32),
            pltpu.VMEM((KB,), jnp.float32),
        ],
    )
    def hist_kernel(dst_hbm, zeros_hbm, ones_hbm, deg_out,
                    deg_sp, dst_buf, ones_v):
        c = lax.axis_index("c")
        s = lax.axis_index("s")
        wid = c * NS + s
        sl = pl.ds(s * rows_per_tile, rows_per_tile)
        pltpu.sync_copy(zeros_hbm.at[sl], deg_sp.at[sl])
        pltpu.sync_copy(ones_hbm, ones_v)
        plsc.subcore_barrier()

        def outer(g, carry):
            pltpu.sync_copy(dst_hbm.at[wid, g], dst_buf)
            pltpu.sync_copy(ones_v, deg_sp.at[dst_buf], add=True)
            return carry

        lax.fori_loop(0, G, outer, 0)
        plsc.subcore_barrier()
        pltpu.sync_copy(deg_sp.at[sl], deg_out.at[c, sl])

    return hist_kernel(dstp, zeros1, ones_kb)


def _sc_edges(srcp, dstp, degp, xt, zeros1, n_pad, G, KB):
    rows_per_tile = n_pad // NS
    mesh = plsc.VectorSubcoreMesh(core_axis_name="c", subcore_axis_name="s")

    @functools.partial(
        pl.kernel,
        out_type=jax.ShapeDtypeStruct((NC, 4, n_pad), jnp.float32),
        mesh=mesh,
        compiler_params=pltpu.CompilerParams(needs_layout_passes=False),
        scratch_types=[
            pltpu.VMEM_SHARED((n_pad,), jnp.float32),
            pltpu.VMEM_SHARED((n_pad,), jnp.float32),
            pltpu.VMEM_SHARED((n_pad,), jnp.float32),
            pltpu.VMEM_SHARED((n_pad,), jnp.float32),
            pltpu.VMEM_SHARED((n_pad,), jnp.float32),
            pltpu.VMEM_SHARED((n_pad,), jnp.float32),
            pltpu.VMEM_SHARED((n_pad,), jnp.float32),
            pltpu.VMEM_SHARED((n_pad,), jnp.float32),
            pltpu.VMEM((KB,), jnp.int32),
            pltpu.VMEM((KB,), jnp.int32),
            pltpu.VMEM((KB,), jnp.float32),
            pltpu.VMEM((KB,), jnp.float32),
            pltpu.VMEM((KB,), jnp.float32),
            pltpu.VMEM((KB,), jnp.float32),
            pltpu.VMEM((rows_per_tile,), jnp.float32),
            pltpu.VMEM((rows_per_tile,), jnp.float32),
            pltpu.SemaphoreType.DMA,
            pltpu.SemaphoreType.DMA,
            pltpu.SemaphoreType.DMA,
            pltpu.SemaphoreType.DMA,
            pltpu.SemaphoreType.DMA,
            pltpu.SemaphoreType.DMA,
            pltpu.SemaphoreType.DMA,
            pltpu.SemaphoreType.DMA,
        ],
    )
    def edge_kernel(src_hbm, dst_hbm, degp_hbm, xt_hbm, z_hbm, acc_out,
                    a0, a1, a2, a3, w0_sp, w1_sp, w2_sp, w3_sp,
                    src_buf, dst_buf, cv0, cv1, cv2, cv3, disv, xv,
                    g0, g1, g2, g3, s0, s1, s2, s3):
        c = lax.axis_index("c")
        s = lax.axis_index("s")
        wid = c * NS + s
        row0 = s * rows_per_tile
        sl = pl.ds(row0, rows_per_tile)
        for a in (a0, a1, a2, a3):
            pltpu.sync_copy(z_hbm.at[sl], a.at[sl])
        pltpu.sync_copy(degp_hbm.at[0, sl], disv)
        pltpu.sync_copy(degp_hbm.at[1, sl], xv)

        def rsqrt_step(i, carry):
            ds16 = pl.ds(i * L, L)
            deg = disv[ds16] + xv[ds16] + 1.0
            iy = jnp.int32(0x5F3759DF) - lax.shift_right_logical(
                plsc.bitcast(deg, jnp.int32), 1)
            y = plsc.bitcast(iy, jnp.float32)
            y = y * (1.5 - 0.5 * deg * y * y)
            y = y * (1.5 - 0.5 * deg * y * y)
            y = y * (1.5 - 0.5 * deg * y * y)
            disv[ds16] = y
            return carry

        lax.fori_loop(0, rows_per_tile // L, rsqrt_step, 0)
        for k, w_sp in enumerate((w0_sp, w1_sp, w2_sp, w3_sp)):
            pltpu.sync_copy(xt_hbm.at[k, sl], xv)

            def scale_step(i, carry):
                ds16 = pl.ds(i * L, L)
                xv[ds16] = xv[ds16] * disv[ds16]
                return carry

            lax.fori_loop(0, rows_per_tile // L, scale_step, 0)
            pltpu.sync_copy(xv, w_sp.at[sl])
        plsc.subcore_barrier()

        cols = ((w0_sp, a0, cv0, g0, s0), (w1_sp, a1, cv1, g1, s1),
                (w2_sp, a2, cv2, g2, s2), (w3_sp, a3, cv3, g3, s3))

        def outer(g, carry):
            pltpu.sync_copy(src_hbm.at[wid, g], src_buf)
            pltpu.sync_copy(dst_hbm.at[wid, g], dst_buf)
            gds = [pltpu.async_copy(w_sp.at[src_buf], cv, gsem)
                   for (w_sp, a, cv, gsem, ssem) in cols]
            sds = []
            for gd, (w_sp, a, cv, gsem, ssem) in zip(gds, cols):
                gd.wait()
                sds.append(pltpu.async_copy(cv, a.at[dst_buf], ssem, add=True))
            for sd in sds:
                sd.wait()
            return carry

        lax.fori_loop(0, G, outer, 0)
        plsc.subcore_barrier()
        for k, a in enumerate((a0, a1, a2, a3)):
            pltpu.sync_copy(a.at[sl], acc_out.at[c, k, sl])

    return edge_kernel(srcp, dstp, degp, xt, zeros1)


def _tc_final(accp, degp, xt, W, b, n, n_pad):
    def body(accp_ref, degp_ref, xt_ref, w_ref, b_ref, out_ref):
        deg = degp_ref[0, :] + degp_ref[1, :] + 1.0
        dis = lax.rsqrt(deg)
        st = dis[None, :] * (accp_ref[0] + accp_ref[1]) \
            + (dis * dis)[None, :] * xt_ref[...]
        out_ref[...] = (
            lax.dot_general(st, w_ref[...], (((0,), (0,)), ((), ())),
                            preferred_element_type=jnp.float32)
            + b_ref[...][None, :]
        )

    return pl.pallas_call(
        body,
        grid=(pl.cdiv(n, C),),
        in_specs=[
            pl.BlockSpec((NC, 4, C), lambda i: (0, 0, i)),
            pl.BlockSpec((NC, C), lambda i: (0, i)),
            pl.BlockSpec((4, C), lambda i: (0, i)),
            pl.BlockSpec((4, 128), lambda i: (0, 0)),
            pl.BlockSpec((128,), lambda i: (0,)),
        ],
        out_specs=pl.BlockSpec((C, 128), lambda i: (i, 0)),
        out_shape=jax.ShapeDtypeStruct((n, W.shape[1]), jnp.float32),
    )(accp, degp, xt, W, b)


def kernel(x, edge_index, W, b):
    n = x.shape[0]
    e = edge_index.shape[1]
    n_pad = pl.cdiv(n + 1, C) * C
    per_w = pl.cdiv(e, NW * B) * B
    total = per_w // B
    K = _pick_k(total)
    G = total // K
    KB = K * B
    pad_e = NW * per_w - e

    fill = jnp.full((pad_e,), n, dtype=jnp.int32)
    srcp = jnp.concatenate([edge_index[0], fill]).reshape(NW, G, KB)
    dstp = jnp.concatenate([edge_index[1], fill]).reshape(NW, G, KB)
    xt = jnp.zeros((4, n_pad), jnp.float32).at[:, :n].set(x.T)
    zeros1 = jnp.zeros((n_pad,), jnp.float32)
    ones_kb = jnp.ones((KB,), jnp.float32)

    degp = _sc_histogram(dstp, zeros1, ones_kb, n_pad, G, KB)
    accp = _sc_edges(srcp, dstp, degp, xt, zeros1, n_pad, G, KB)
    return _tc_final(accp, degp, xt, W, b, n, n_pad)
